# baseline (speedup 1.0000x reference)
import jax
import jax.numpy as jnp
from jax import lax
from jax.experimental import pallas as pl
from jax.experimental.pallas import tpu as pltpu

TS = 16
B, N, H, W = 2, 8, 384, 384
C = 96
K = 128
TH = H // TS
TW = W // TS
GW = 3
NG = TH * GW


def _body(F_ref, act8_ref, ls_ref, mask_ref, ff_ref, out_ref):
    b = pl.program_id(0)
    th = pl.program_id(1)
    wt = pl.program_id(2)
    bits = act8_ref[b, th * GW + wt]

    @pl.when(bits == 0)
    def _():
        out_ref[0] = mask_ref[0]

    @pl.when(bits != 0)
    def _():
        x = ls_ref[0]
        strength = jnp.maximum(x, 0.0) + jnp.log(1.0 + jnp.exp(-jnp.abs(x)))
        sig = jnp.tanh(jnp.sum(ff_ref[0], axis=0) * (1.0 / C))
        j = jax.lax.broadcasted_iota(jnp.int32, (TS, 128), 1) // TS
        active = ((bits >> j) & 1) == 1
        delta = jnp.where(active, strength * sig, 0.0)
        out_ref[0] = mask_ref[0] + delta[None, :, :]


def kernel(mask_logits, ff_highres_features, log_strength, active_tile_indices):
    idx = jnp.asarray(active_tile_indices, jnp.int32)
    ls = jnp.asarray(log_strength, jnp.float32).reshape(1)

    bb = jnp.arange(B, dtype=jnp.int32)[:, None]
    tile_act = (
        jnp.zeros((B, TH * TW), jnp.int32).at[bb, idx].max(1)
    )
    grouped = tile_act.reshape(B, TH, GW, 8)
    act8 = jnp.sum(grouped << jnp.arange(8, dtype=jnp.int32), axis=-1).reshape(
        B, NG
    )
    s_iota = jnp.arange(NG, dtype=jnp.int32)[None, :]
    fetch = lax.cummax(jnp.where(act8 > 0, s_iota, 0), axis=1)

    grid_spec = pltpu.PrefetchScalarGridSpec(
        num_scalar_prefetch=2,
        grid=(B, TH, GW),
        in_specs=[
            pl.BlockSpec(memory_space=pltpu.SMEM),
            pl.BlockSpec((1, N, TS, 128), lambda b, th, wt, F, a8: (b, 0, th, wt)),
            pl.BlockSpec(
                (1, C, TS, 128),
                lambda b, th, wt, F, a8: (
                    b,
                    0,
                    F[b, th * GW + wt] // GW,
                    F[b, th * GW + wt] % GW,
                ),
            ),
        ],
        out_specs=pl.BlockSpec(
            (1, N, TS, 128), lambda b, th, wt, F, a8: (b, 0, th, wt)
        ),
    )
    return pl.pallas_call(
        _body,
        grid_spec=grid_spec,
        out_shape=jax.ShapeDtypeStruct((B, N, H, W), jnp.float32),
    )(fetch, act8, ls, mask_logits, ff_highres_features)


# speedup vs baseline: 3.2956x; 3.2956x over previous
import jax
import jax.numpy as jnp
from jax.experimental import pallas as pl
from jax.experimental.pallas import tpu as pltpu

TS = 16
B, N, H, W = 2, 8, 384, 384
C = 96
K = 128
TW = W // TS
BH = 32
NBLK = H // BH


def _dense_body(idx_ref, ls_ref, mask_ref, ff_ref, out_ref):
    hb = pl.program_id(1)
    x = ls_ref[0]
    strength = jnp.maximum(x, 0.0) + jnp.log(1.0 + jnp.exp(-jnp.abs(x)))
    ffb = ff_ref[0]
    sig = jnp.tanh(jnp.sum(ffb, axis=0) * (1.0 / C))
    trow = hb * (BH // TS) + jax.lax.broadcasted_iota(jnp.int32, (BH, W), 0) // TS
    tcol = trow * TW + jax.lax.broadcasted_iota(jnp.int32, (BH, W), 1) // TS
    active = jnp.zeros((BH, W), dtype=jnp.bool_)
    for k in range(K):
        active = active | (tcol == idx_ref[0, 0, k])
    delta = jnp.where(active, strength * sig, 0.0)
    out_ref[0] = mask_ref[0] + delta[None, :, :]


def kernel(mask_logits, ff_highres_features, log_strength, active_tile_indices):
    idx = jnp.asarray(active_tile_indices, jnp.int32).reshape(B, 1, K)
    ls = jnp.asarray(log_strength, jnp.float32).reshape(1)
    return pl.pallas_call(
        _dense_body,
        grid=(B, NBLK),
        in_specs=[
            pl.BlockSpec((1, 1, K), lambda b, hb: (b, 0, 0), memory_space=pltpu.SMEM),
            pl.BlockSpec(memory_space=pltpu.SMEM),
            pl.BlockSpec((1, N, BH, W), lambda b, hb: (b, 0, hb, 0)),
            pl.BlockSpec((1, C, BH, W), lambda b, hb: (b, 0, hb, 0)),
        ],
        out_specs=pl.BlockSpec((1, N, BH, W), lambda b, hb: (b, 0, hb, 0)),
        out_shape=jax.ShapeDtypeStruct((B, N, H, W), jnp.float32),
    )(idx, ls, mask_logits, ff_highres_features)
